# baseline (speedup 1.0000x reference)
import math

import jax
import jax.numpy as jnp
import numpy as np
from jax.experimental import pallas as pl

_N = 1024
_R = 6
_W = 8
_H = 64
_RMIN = 0.1
_RMAX = 10.0
_FC_IN = _R * _W * _H
_FC_OUT = 64
_LOG_RMAX_BY_RMIN = math.log(int(_RMAX / float(_RMIN)))

_TILE = 256
_GROUP = 16


def _social_pool_kernel(x_col, y_col, x_row, y_row, hidden, w_fc, b_fc, out):
    xi = x_col[:, 0:1]
    yi = y_col[:, 0:1]
    xj = x_row[0:1, :]
    yj = y_row[0:1, :]

    x_diff = xj - xi
    y_diff = yj - yi
    d2 = x_diff * x_diff + y_diff * y_diff
    r = jnp.sqrt(d2)

    ring_f = jnp.floor((_R - 1) * (jnp.log(r / _RMIN) / _LOG_RMAX_BY_RMIN))
    valid = (r >= _RMIN) & (ring_f < _R)
    ring_c = jnp.clip(ring_f, 0.0, _R - 1)

    theta = jnp.arctan2(y_diff, x_diff)
    wedge_f = jnp.trunc(theta * _W / (2.0 * np.pi) + (_W // 2 - 1))

    h = hidden[...]
    cellb = jnp.where(valid, ring_c * _W + wedge_f,
                      float(_R * _W)).astype(jnp.bfloat16)
    means = []
    for g0 in range(0, _R * _W, _GROUP):
        mg = jnp.concatenate(
            [jnp.where(cellb == jnp.bfloat16(c),
                       jnp.bfloat16(1.0), jnp.bfloat16(0.0))
             for c in range(g0, g0 + _GROUP)], axis=0)
        sg = jnp.dot(mg, h, preferred_element_type=jnp.float32)
        for k in range(_GROUP):
            s = sg[k * _TILE:(k + 1) * _TILE, :]
            cnt = s[:, _H:_H + 1]
            sc = s[:, :_H]
            rec = 1.0 / jnp.maximum(cnt, 1.0)
            means.append((sc * rec).astype(jnp.bfloat16))

    mean_flat = jnp.concatenate(means, axis=1)
    acc = jnp.dot(mean_flat, w_fc[...], preferred_element_type=jnp.float32)
    out[...] = jnp.maximum(acc + b_fc[0:1, :], 0.0)


def kernel(ypred, hidden, W_fc, b_fc):
    yd = jax.lax.stop_gradient(ypred)
    x_col = yd[:, 0:1]
    y_col = yd[:, 1:2]
    x_row = yd[:, 0].reshape(1, _N)
    y_row = yd[:, 1].reshape(1, _N)
    b2 = b_fc.reshape(1, _FC_OUT)
    hidden_aug = jnp.concatenate(
        [hidden, jnp.ones((_N, 1), jnp.float32)], axis=1
    ).astype(jnp.bfloat16)
    w_bf = W_fc.astype(jnp.bfloat16)

    grid = (_N // _TILE,)
    return pl.pallas_call(
        _social_pool_kernel,
        grid=grid,
        in_specs=[
            pl.BlockSpec((_TILE, 1), lambda t: (t, 0)),
            pl.BlockSpec((_TILE, 1), lambda t: (t, 0)),
            pl.BlockSpec((1, _N), lambda t: (0, 0)),
            pl.BlockSpec((1, _N), lambda t: (0, 0)),
            pl.BlockSpec((_N, _H + 1), lambda t: (0, 0)),
            pl.BlockSpec((_FC_IN, _FC_OUT), lambda t: (0, 0)),
            pl.BlockSpec((1, _FC_OUT), lambda t: (0, 0)),
        ],
        out_specs=pl.BlockSpec((_TILE, _FC_OUT), lambda t: (t, 0)),
        out_shape=jax.ShapeDtypeStruct((_N, _FC_OUT), jnp.float32),
    )(x_col, y_col, x_row, y_row, hidden_aug, w_bf, b2)


# speedup vs baseline: 1.0767x; 1.0767x over previous
import math

import jax
import jax.numpy as jnp
import numpy as np
from jax.experimental import pallas as pl

_N = 1024
_R = 6
_W = 8
_H = 64
_RMIN = 0.1
_RMAX = 10.0
_FC_IN = _R * _W * _H
_FC_OUT = 64
_LOG_RMAX_BY_RMIN = math.log(int(_RMAX / float(_RMIN)))

_TILE = 256
_GROUP = 12


def _social_pool_kernel(x_col, y_col, x_row, y_row, hidden, w_fc, b_fc, out):
    xi = x_col[:, 0:1]
    yi = y_col[:, 0:1]
    xj = x_row[0:1, :]
    yj = y_row[0:1, :]

    x_diff = xj - xi
    y_diff = yj - yi
    d2 = x_diff * x_diff + y_diff * y_diff
    r = jnp.sqrt(d2)

    ring_f = jnp.floor((_R - 1) * (jnp.log(r / _RMIN) / _LOG_RMAX_BY_RMIN))
    valid = (r >= _RMIN) & (ring_f < _R)
    ring_c = jnp.clip(ring_f, 0.0, _R - 1)

    theta = jnp.arctan2(y_diff, x_diff)
    wedge_f = jnp.trunc(theta * _W / (2.0 * np.pi) + (_W // 2 - 1))

    h = hidden[...]
    cellb = jnp.where(valid, ring_c * _W + wedge_f,
                      float(_R * _W)).astype(jnp.bfloat16)
    means = []
    for g0 in range(0, _R * _W, _GROUP):
        mg = jnp.concatenate(
            [jnp.where(cellb == jnp.bfloat16(c),
                       jnp.bfloat16(1.0), jnp.bfloat16(0.0))
             for c in range(g0, g0 + _GROUP)], axis=0)
        sg = jnp.dot(mg, h, preferred_element_type=jnp.float32)
        for k in range(_GROUP):
            s = sg[k * _TILE:(k + 1) * _TILE, :]
            cnt = s[:, _H:_H + 1]
            sc = s[:, :_H]
            rec = 1.0 / jnp.maximum(cnt, 1.0)
            means.append((sc * rec).astype(jnp.bfloat16))

    mean_flat = jnp.concatenate(means, axis=1)
    acc = jnp.dot(mean_flat, w_fc[...], preferred_element_type=jnp.float32)
    out[...] = jnp.maximum(acc + b_fc[0:1, :], 0.0)


def kernel(ypred, hidden, W_fc, b_fc):
    yd = jax.lax.stop_gradient(ypred)
    x_col = yd[:, 0:1]
    y_col = yd[:, 1:2]
    x_row = yd[:, 0].reshape(1, _N)
    y_row = yd[:, 1].reshape(1, _N)
    b2 = b_fc.reshape(1, _FC_OUT)
    hidden_aug = jnp.concatenate(
        [hidden, jnp.ones((_N, 1), jnp.float32)], axis=1
    ).astype(jnp.bfloat16)
    w_bf = W_fc.astype(jnp.bfloat16)

    grid = (_N // _TILE,)
    return pl.pallas_call(
        _social_pool_kernel,
        grid=grid,
        in_specs=[
            pl.BlockSpec((_TILE, 1), lambda t: (t, 0)),
            pl.BlockSpec((_TILE, 1), lambda t: (t, 0)),
            pl.BlockSpec((1, _N), lambda t: (0, 0)),
            pl.BlockSpec((1, _N), lambda t: (0, 0)),
            pl.BlockSpec((_N, _H + 1), lambda t: (0, 0)),
            pl.BlockSpec((_FC_IN, _FC_OUT), lambda t: (0, 0)),
            pl.BlockSpec((1, _FC_OUT), lambda t: (0, 0)),
        ],
        out_specs=pl.BlockSpec((_TILE, _FC_OUT), lambda t: (t, 0)),
        out_shape=jax.ShapeDtypeStruct((_N, _FC_OUT), jnp.float32),
    )(x_col, y_col, x_row, y_row, hidden_aug, w_bf, b2)
